# baseline (speedup 1.0000x reference)
import jax
import jax.numpy as jnp
from jax.experimental import pallas as pl

_B, _K = 16384, 64
_NB = 8
_CH = (_B // 2) // _NB


def _body(u_ref, v_ref, out0_ref, out1_ref):
    prod = u_ref[...] * v_ref[...]
    s0 = jnp.sum(prod[:, :_K], axis=1)
    s1 = jnp.sum(prod[:, _K:], axis=1)
    out0_ref[...] = s0.reshape(_CH // 128, 128)
    out1_ref[...] = s1.reshape(_CH // 128, 128)


def kernel(gu, gi):
    u = gu.reshape(_B // 2, 2 * _K)
    v = gi.reshape(_B // 2, 2 * _K)
    out0, out1 = pl.pallas_call(
        _body,
        grid=(_NB,),
        in_specs=[
            pl.BlockSpec((_CH, 2 * _K), lambda i: (i, 0)),
            pl.BlockSpec((_CH, 2 * _K), lambda i: (i, 0)),
        ],
        out_specs=[
            pl.BlockSpec((_CH // 128, 128), lambda i: (i, 0)),
            pl.BlockSpec((_CH // 128, 128), lambda i: (i, 0)),
        ],
        out_shape=[
            jax.ShapeDtypeStruct((_B // 256, 128), jnp.float32),
            jax.ShapeDtypeStruct((_B // 256, 128), jnp.float32),
        ],
    )(u, v)
    pair = jnp.stack([out0.reshape(_B // 2), out1.reshape(_B // 2)], axis=1)
    return pair.reshape(_B)


# speedup vs baseline: 6.0734x; 6.0734x over previous
import jax
import jax.numpy as jnp
from jax.experimental import pallas as pl

_B, _K = 16384, 64
_NB = 8
_CB = _B // _NB


def _body(u_ref, v_ref, out_ref):
    s = jnp.sum(u_ref[...] * v_ref[...], axis=0)
    out_ref[...] = s.reshape(_CB // 128, 128)


def kernel(gu, gi):
    out = pl.pallas_call(
        _body,
        grid=(_NB,),
        in_specs=[
            pl.BlockSpec((_K, _CB), lambda i: (0, i)),
            pl.BlockSpec((_K, _CB), lambda i: (0, i)),
        ],
        out_specs=pl.BlockSpec((_CB // 128, 128), lambda i: (i, 0)),
        out_shape=jax.ShapeDtypeStruct((_B // 128, 128), jnp.float32),
    )(gu.T, gi.T)
    return out.reshape(_B)


# speedup vs baseline: 8.8179x; 1.4519x over previous
import jax
import jax.numpy as jnp
from jax.experimental import pallas as pl
from jax.experimental.pallas import tpu as pltpu

_B, _K = 16384, 64
_NB = 8
_CB = _B // _NB


def _body(u_ref, v_ref, out_ref):
    s = jnp.sum(u_ref[...] * v_ref[...], axis=0)
    out_ref[...] = s.reshape(_B // 128, 128)


def kernel(gu, gi):
    out = pl.pallas_call(
        _body,
        in_specs=[
            pl.BlockSpec(memory_space=pltpu.VMEM),
            pl.BlockSpec(memory_space=pltpu.VMEM),
        ],
        out_specs=pl.BlockSpec(memory_space=pltpu.VMEM),
        out_shape=jax.ShapeDtypeStruct((_B // 128, 128), jnp.float32),
    )(gu.T, gi.T)
    return out.reshape(_B)


# speedup vs baseline: 9.5362x; 1.0815x over previous
import jax
import jax.numpy as jnp
from jax.experimental import pallas as pl
from jax.experimental.pallas import tpu as pltpu

_B, _K = 16384, 64
_NCH = 8
_CB = _B // _NCH


def _body(u_hbm, v_hbm, out_ref, u_v, v_v, sems):
    copies = []
    for c in range(_NCH):
        cu = pltpu.make_async_copy(
            u_hbm.at[:, pl.ds(c * _CB, _CB)],
            u_v.at[:, pl.ds(c * _CB, _CB)],
            sems.at[0, c],
        )
        cv = pltpu.make_async_copy(
            v_hbm.at[:, pl.ds(c * _CB, _CB)],
            v_v.at[:, pl.ds(c * _CB, _CB)],
            sems.at[1, c],
        )
        cu.start()
        cv.start()
        copies.append((cu, cv))
    for c in range(_NCH):
        cu, cv = copies[c]
        cu.wait()
        cv.wait()
        s = jnp.sum(
            u_v[:, pl.ds(c * _CB, _CB)] * v_v[:, pl.ds(c * _CB, _CB)], axis=0
        )
        out_ref[pl.ds(c * (_CB // 128), _CB // 128), :] = s.reshape(_CB // 128, 128)


def kernel(gu, gi):
    out = pl.pallas_call(
        _body,
        in_specs=[
            pl.BlockSpec(memory_space=pltpu.HBM),
            pl.BlockSpec(memory_space=pltpu.HBM),
        ],
        out_specs=pl.BlockSpec(memory_space=pltpu.VMEM),
        out_shape=jax.ShapeDtypeStruct((_B // 128, 128), jnp.float32),
        scratch_shapes=[
            pltpu.VMEM((_K, _B), jnp.float32),
            pltpu.VMEM((_K, _B), jnp.float32),
            pltpu.SemaphoreType.DMA((2, _NCH)),
        ],
    )(gu.T, gi.T)
    return out.reshape(_B)
